# baseline (speedup 1.0000x reference)
import dataclasses
import functools

import jax
import jax.numpy as jnp
from jax import lax
from jax.experimental import pallas as pl
from jax.experimental.pallas import tpu as pltpu
from jax.experimental.pallas import tpu_sc as plsc

N = 10000
NP = 10240
D = 128
E = 320000
CH = 128
NC = 2
NS = 16
NW = NC * NS
CHUNKS = 2560
EPAD = CHUNKS * CH
CPW = CHUNKS // NW
DUMMY_DST = N
ZERO_SRC = N + 8
RPS = NP // NS
BLK = 1024


def _sc_mesh():
    return plsc.VectorSubcoreMesh(core_axis_name="c", subcore_axis_name="s")


EPW = EPAD // NW


def _sc_compiler_params():
    cp = pltpu.CompilerParams()
    if "needs_layout_passes" in pltpu.CompilerParams.__dataclass_fields__:
        cp = dataclasses.replace(cp, needs_layout_passes=False)
    return cp


@functools.partial(pl.kernel,
                   compiler_params=_sc_compiler_params(),
                   out_type=jax.ShapeDtypeStruct((NW, NP), jnp.float32),
                   mesh=_sc_mesh(),
                   scratch_types=[pltpu.VMEM((EPW,), jnp.int32),
                                  pltpu.VMEM((NP,), jnp.float32)])
def _sc_degree(dst_hbm, out_hbm, idx_v, hist_v):
    core = lax.axis_index("c")
    sid = lax.axis_index("s")
    wid = sid * NC + core

    @pl.loop(0, NP, step=16)
    def _(i):
        hist_v[pl.ds(i, 16)] = jnp.zeros((16,), jnp.float32)

    pltpu.sync_copy(dst_hbm.at[pl.ds(wid * EPW, EPW)], idx_v)
    ones = jnp.ones((16,), jnp.float32)

    @pl.loop(0, EPW, step=16)
    def _(e):
        idx = idx_v[pl.ds(e, 16)]
        plsc.addupdate_scatter(hist_v, [idx], ones)

    pltpu.sync_copy(hist_v, out_hbm.at[wid])


QC = 16


QB = 2 * QC


@functools.partial(pl.kernel,
                   out_type=jax.ShapeDtypeStruct((NC, NP, D), jnp.float32),
                   mesh=_sc_mesh(),
                   scratch_types=[pltpu.VMEM((QB, CH), jnp.int32),
                                  pltpu.VMEM((QB, CH), jnp.int32),
                                  pltpu.VMEM((2, CH, D), jnp.float32),
                                  pltpu.VMEM_SHARED((NP, D), jnp.float32),
                                  pltpu.SemaphoreType.DMA,
                                  pltpu.SemaphoreType.DMA,
                                  pltpu.SemaphoreType.DMA,
                                  pltpu.SemaphoreType.DMA])
def _sc_aggregate(g_hbm, src_hbm, dst_hbm, out_hbm,
                  sidx_v, didx_v, bufs, acc, gs0, gs1, ss0, ss1):
    core = lax.axis_index("c")
    sid = lax.axis_index("s")
    wid = sid * NC + core
    r0 = sid * RPS

    @pl.loop(0, CH)
    def _(i):
        @pl.loop(0, D, step=16)
        def _(j):
            bufs[0, i, pl.ds(j, 16)] = jnp.zeros((16,), jnp.float32)

    @pl.loop(0, RPS, step=CH)
    def _(r):
        pltpu.sync_copy(bufs.at[0], acc.at[pl.ds(r0 + r, CH)])

    base0 = pl.multiple_of(wid * CPW, 8)
    pltpu.sync_copy(src_hbm.at[pl.ds(base0, QB)], sidx_v)
    pltpu.sync_copy(dst_hbm.at[pl.ds(base0, QB)], didx_v)
    pltpu.async_copy(g_hbm.at[sidx_v.at[0]], bufs.at[0], gs0)

    plsc.subcore_barrier()

    @pl.loop(0, CPW, step=2)
    def _(c0):
        for b in range(2):
            c = c0 + b
            gsem = gs0 if b == 0 else gs1
            gsem_n = gs1 if b == 0 else gs0
            ssemb = ss0 if b == 0 else ss1
            ssemo = ss1 if b == 0 else ss0
            pltpu.make_async_copy(g_hbm.at[pl.ds(0, CH)],
                                  bufs.at[b], gsem).wait()
            nxt = c + 1

            @pl.when(c >= 1)
            def _():
                pltpu.make_async_copy(bufs.at[1 - b],
                                      acc.at[pl.ds(0, CH)], ssemo).wait()

            @pl.when((lax.rem(c, QC) == 0) & (c >= 1) & (c + QC < CPW))
            def _():
                base = pl.multiple_of(wid * CPW + c + QC, 8)
                roff = pl.multiple_of(lax.rem(c + QC, QB), 8)
                pltpu.sync_copy(src_hbm.at[pl.ds(base, QC)],
                                sidx_v.at[pl.ds(roff, QC)])
                pltpu.sync_copy(dst_hbm.at[pl.ds(base, QC)],
                                didx_v.at[pl.ds(roff, QC)])

            @pl.when(nxt < CPW)
            def _():
                pltpu.async_copy(g_hbm.at[sidx_v.at[lax.rem(nxt, QB)]],
                                 bufs.at[1 - b], gsem_n)

            pltpu.async_copy(bufs.at[b], acc.at[didx_v.at[lax.rem(c, QB)]],
                             ssemb, add=True)

    pltpu.make_async_copy(bufs.at[1], acc.at[pl.ds(0, CH)], ss1).wait()

    plsc.subcore_barrier()

    @pl.loop(0, RPS, step=CH)
    def _(r):
        pltpu.sync_copy(acc.at[pl.ds(r0 + r, CH)],
                        out_hbm.at[core, pl.ds(r0 + r, CH)])


def _first_body(x_ref, w_ref, d_ref, g_ref, dv_ref):
    deg = lax.dot_general(d_ref[...], jnp.ones((NW, 1), jnp.float32),
                          (((0,), (0,)), ((), ())),
                          preferred_element_type=jnp.float32) + 1.0
    dinv = lax.rsqrt(deg)
    dv_ref[...] = jnp.broadcast_to(dinv, (BLK, 16))
    h = jnp.dot(x_ref[...], w_ref[...], preferred_element_type=jnp.float32)
    g_ref[...] = h * dinv


def _tc_first(x_p, w, dall):
    return pl.pallas_call(
        _first_body,
        grid=(NP // BLK,),
        in_specs=[pl.BlockSpec((BLK, D), lambda i: (i, 0)),
                  pl.BlockSpec((D, D), lambda i: (0, 0)),
                  pl.BlockSpec((NW, BLK), lambda i: (0, i))],
        out_specs=[pl.BlockSpec((BLK, D), lambda i: (i, 0)),
                   pl.BlockSpec((BLK, 16), lambda i: (i, 0))],
        out_shape=[jax.ShapeDtypeStruct((NP, D), jnp.float32),
                   jax.ShapeDtypeStruct((NP, 16), jnp.float32)],
    )(x_p, w, dall)


def _mid_body(a0_ref, a1_ref, gp_ref, dv_ref, b_ref, w_ref, o_ref):
    dinv = dv_ref[...][:, 0:1]
    s = (a0_ref[...] + a1_ref[...] + gp_ref[...]) * dinv + b_ref[...]
    h = jnp.maximum(s, 0.0)
    o_ref[...] = jnp.dot(h, w_ref[...],
                         preferred_element_type=jnp.float32) * dinv


def _tc_mid(a0, a1, gp, dv, b, w):
    return pl.pallas_call(
        _mid_body,
        grid=(NP // BLK,),
        in_specs=[pl.BlockSpec((BLK, D), lambda i: (i, 0)),
                  pl.BlockSpec((BLK, D), lambda i: (i, 0)),
                  pl.BlockSpec((BLK, D), lambda i: (i, 0)),
                  pl.BlockSpec((BLK, 16), lambda i: (i, 0)),
                  pl.BlockSpec((1, D), lambda i: (0, 0)),
                  pl.BlockSpec((D, D), lambda i: (0, 0))],
        out_specs=pl.BlockSpec((BLK, D), lambda i: (i, 0)),
        out_shape=jax.ShapeDtypeStruct((NP, D), jnp.float32),
    )(a0, a1, gp, dv, b, w)


def _last_body(a0_ref, a1_ref, gp_ref, dv_ref, b_ref, o_ref):
    dinv = dv_ref[...][:, 0:1]
    o_ref[...] = (a0_ref[...] + a1_ref[...] + gp_ref[...]) * dinv + b_ref[...]


def _tc_last(a0, a1, gp, dv, b):
    return pl.pallas_call(
        _last_body,
        grid=(NP // BLK,),
        in_specs=[pl.BlockSpec((BLK, D), lambda i: (i, 0)),
                  pl.BlockSpec((BLK, D), lambda i: (i, 0)),
                  pl.BlockSpec((BLK, D), lambda i: (i, 0)),
                  pl.BlockSpec((BLK, 16), lambda i: (i, 0)),
                  pl.BlockSpec((1, D), lambda i: (0, 0))],
        out_specs=pl.BlockSpec((BLK, D), lambda i: (i, 0)),
        out_shape=jax.ShapeDtypeStruct((NP, D), jnp.float32),
    )(a0, a1, gp, dv, b)


def kernel(x, edge_index, W0, b0, W1, b1, W2, b2):
    ei = edge_index.astype(jnp.int32)
    pad = EPAD - E
    pad_rows = DUMMY_DST + jnp.arange(pad, dtype=jnp.int32) % (NP - N)
    src = jnp.concatenate([ei[0], pad_rows])
    dst = jnp.concatenate([ei[1], pad_rows])
    srcm = src.reshape(CHUNKS, CH)
    dstm = dst.reshape(CHUNKS, CH)
    x_p = jnp.pad(x, ((0, NP - N), (0, 0)))
    b0r = b0.reshape(1, D)
    b1r = b1.reshape(1, D)
    b2r = b2.reshape(1, D)

    dall = _sc_degree(dst)
    g0, dv = _tc_first(x_p, W0, dall)

    a = _sc_aggregate(g0, srcm, dstm)
    g1 = _tc_mid(a[0], a[1], g0, dv, b0r, W1)

    a = _sc_aggregate(g1, srcm, dstm)
    g2 = _tc_mid(a[0], a[1], g1, dv, b1r, W2)

    a = _sc_aggregate(g2, srcm, dstm)
    out = _tc_last(a[0], a[1], g2, dv, b2r)
    return out[:N]


# speedup vs baseline: 1.0475x; 1.0475x over previous
import dataclasses
import functools

import jax
import jax.numpy as jnp
from jax import lax
from jax.experimental import pallas as pl
from jax.experimental.pallas import tpu as pltpu
from jax.experimental.pallas import tpu_sc as plsc

N = 10000
NP = 10240
D = 128
E = 320000
CH = 128
NC = 2
NS = 16
NW = NC * NS
CHUNKS = 2560
EPAD = CHUNKS * CH
CPW = CHUNKS // NW
DUMMY_DST = N
ZERO_SRC = N + 8
RPS = NP // NS
BLK = 1024


def _sc_mesh():
    return plsc.VectorSubcoreMesh(core_axis_name="c", subcore_axis_name="s")


EPW = EPAD // NW


def _sc_compiler_params():
    cp = pltpu.CompilerParams()
    if "needs_layout_passes" in pltpu.CompilerParams.__dataclass_fields__:
        cp = dataclasses.replace(cp, needs_layout_passes=False)
    return cp


@functools.partial(pl.kernel,
                   compiler_params=_sc_compiler_params(),
                   out_type=jax.ShapeDtypeStruct((NW, NP), jnp.float32),
                   mesh=_sc_mesh(),
                   scratch_types=[pltpu.VMEM((EPW,), jnp.int32),
                                  pltpu.VMEM((NP,), jnp.float32)])
def _sc_degree(dst_hbm, out_hbm, idx_v, hist_v):
    core = lax.axis_index("c")
    sid = lax.axis_index("s")
    wid = sid * NC + core

    @pl.loop(0, NP, step=16)
    def _(i):
        hist_v[pl.ds(i, 16)] = jnp.zeros((16,), jnp.float32)

    pltpu.sync_copy(dst_hbm.at[pl.ds(wid * EPW, EPW)], idx_v)
    ones = jnp.ones((16,), jnp.float32)

    @pl.loop(0, EPW, step=16)
    def _(e):
        idx = idx_v[pl.ds(e, 16)]
        plsc.addupdate_scatter(hist_v, [idx], ones)

    pltpu.sync_copy(hist_v, out_hbm.at[wid])


QC = 16


QB = 2 * QC


@functools.partial(pl.kernel,
                   out_type=jax.ShapeDtypeStruct((NC, NP, D), jnp.float32),
                   mesh=_sc_mesh(),
                   scratch_types=[pltpu.VMEM((QB, CH), jnp.int32),
                                  pltpu.VMEM((QB, CH), jnp.int32),
                                  pltpu.VMEM((2, CH, D), jnp.float32),
                                  pltpu.VMEM_SHARED((NP, D), jnp.float32),
                                  pltpu.SemaphoreType.DMA,
                                  pltpu.SemaphoreType.DMA,
                                  pltpu.SemaphoreType.DMA,
                                  pltpu.SemaphoreType.DMA])
def _sc_aggregate(g_hbm, src_hbm, dst_hbm, out_hbm,
                  sidx_v, didx_v, bufs, acc, gs0, gs1, ss0, ss1):
    core = lax.axis_index("c")
    sid = lax.axis_index("s")
    wid = sid * NC + core
    r0 = sid * RPS

    @pl.loop(0, CH)
    def _(i):
        @pl.loop(0, D, step=16)
        def _(j):
            bufs[0, i, pl.ds(j, 16)] = jnp.zeros((16,), jnp.float32)

    @pl.loop(0, RPS, step=CH)
    def _(r):
        pltpu.sync_copy(bufs.at[0], acc.at[pl.ds(r0 + r, CH)])

    base0 = pl.multiple_of(wid * CPW, 8)
    pltpu.sync_copy(src_hbm.at[pl.ds(base0, QB)], sidx_v)
    pltpu.sync_copy(dst_hbm.at[pl.ds(base0, QB)], didx_v)
    pltpu.async_copy(g_hbm.at[sidx_v.at[0]], bufs.at[0], gs0)

    plsc.subcore_barrier()

    @pl.loop(0, CPW, step=2)
    def _(c0):
        for b in range(2):
            c = c0 + b
            gsem = gs0 if b == 0 else gs1
            gsem_n = gs1 if b == 0 else gs0
            ssemb = ss0 if b == 0 else ss1
            ssemo = ss1 if b == 0 else ss0
            pltpu.make_async_copy(g_hbm.at[pl.ds(0, CH)],
                                  bufs.at[b], gsem).wait()
            nxt = c + 1

            @pl.when(c >= 1)
            def _():
                pltpu.make_async_copy(bufs.at[1 - b],
                                      acc.at[pl.ds(0, CH)], ssemo).wait()

            @pl.when((lax.rem(c, QC) == 0) & (c >= 1) & (c + QC < CPW))
            def _():
                base = pl.multiple_of(wid * CPW + c + QC, 8)
                roff = pl.multiple_of(lax.rem(c + QC, QB), 8)
                pltpu.sync_copy(src_hbm.at[pl.ds(base, QC)],
                                sidx_v.at[pl.ds(roff, QC)])
                pltpu.sync_copy(dst_hbm.at[pl.ds(base, QC)],
                                didx_v.at[pl.ds(roff, QC)])

            @pl.when(nxt < CPW)
            def _():
                pltpu.async_copy(g_hbm.at[sidx_v.at[lax.rem(nxt, QB)]],
                                 bufs.at[1 - b], gsem_n)

            pltpu.async_copy(bufs.at[b], acc.at[didx_v.at[lax.rem(c, QB)]],
                             ssemb, add=True)

    pltpu.make_async_copy(bufs.at[1], acc.at[pl.ds(0, CH)], ss1).wait()

    plsc.subcore_barrier()

    @pl.loop(0, RPS, step=CH)
    def _(r):
        pltpu.sync_copy(acc.at[pl.ds(r0 + r, CH)],
                        out_hbm.at[core, pl.ds(r0 + r, CH)])


def _first_body(x_ref, w_ref, d_ref, g_ref, dv_ref):
    deg = lax.dot_general(d_ref[...], jnp.ones((NW, 1), jnp.float32),
                          (((0,), (0,)), ((), ())),
                          preferred_element_type=jnp.float32) + 1.0
    dinv = lax.rsqrt(deg)
    dv_ref[...] = jnp.broadcast_to(dinv, (BLK, 16))
    h = jnp.dot(x_ref[...], w_ref[...], preferred_element_type=jnp.float32)
    g_ref[...] = h * dinv


def _tc_first(x_p, w, dall):
    return pl.pallas_call(
        _first_body,
        grid=(NP // BLK,),
        in_specs=[pl.BlockSpec((BLK, D), lambda i: (i, 0)),
                  pl.BlockSpec((D, D), lambda i: (0, 0)),
                  pl.BlockSpec((NW, BLK), lambda i: (0, i))],
        out_specs=[pl.BlockSpec((BLK, D), lambda i: (i, 0)),
                   pl.BlockSpec((BLK, 16), lambda i: (i, 0))],
        out_shape=[jax.ShapeDtypeStruct((NP, D), jnp.float32),
                   jax.ShapeDtypeStruct((NP, 16), jnp.float32)],
    )(x_p, w, dall)


def _mid_body(a_ref, gp_ref, dv_ref, b_ref, w_ref, o_ref):
    dinv = dv_ref[...][:, 0:1]
    s = (a_ref[0] + a_ref[1] + gp_ref[...]) * dinv + b_ref[...]
    h = jnp.maximum(s, 0.0)
    o_ref[...] = jnp.dot(h, w_ref[...],
                         preferred_element_type=jnp.float32) * dinv


def _tc_mid(a, gp, dv, b, w):
    return pl.pallas_call(
        _mid_body,
        grid=(NP // BLK,),
        in_specs=[pl.BlockSpec((NC, BLK, D), lambda i: (0, i, 0)),
                  pl.BlockSpec((BLK, D), lambda i: (i, 0)),
                  pl.BlockSpec((BLK, 16), lambda i: (i, 0)),
                  pl.BlockSpec((1, D), lambda i: (0, 0)),
                  pl.BlockSpec((D, D), lambda i: (0, 0))],
        out_specs=pl.BlockSpec((BLK, D), lambda i: (i, 0)),
        out_shape=jax.ShapeDtypeStruct((NP, D), jnp.float32),
    )(a, gp, dv, b, w)


BLKL = 1000


def _last_body(a_ref, gp_ref, dv_ref, b_ref, o_ref):
    dinv = dv_ref[...][:, 0:1]
    o_ref[...] = (a_ref[0] + a_ref[1] + gp_ref[...]) * dinv + b_ref[...]


def _tc_last(a, gp, dv, b):
    return pl.pallas_call(
        _last_body,
        grid=(N // BLKL,),
        in_specs=[pl.BlockSpec((NC, BLKL, D), lambda i: (0, i, 0)),
                  pl.BlockSpec((BLKL, D), lambda i: (i, 0)),
                  pl.BlockSpec((BLKL, 16), lambda i: (i, 0)),
                  pl.BlockSpec((1, D), lambda i: (0, 0))],
        out_specs=pl.BlockSpec((BLKL, D), lambda i: (i, 0)),
        out_shape=jax.ShapeDtypeStruct((N, D), jnp.float32),
    )(a, gp, dv, b)


def kernel(x, edge_index, W0, b0, W1, b1, W2, b2):
    ei = edge_index.astype(jnp.int32)
    pad = EPAD - E
    pad_rows = DUMMY_DST + jnp.arange(pad, dtype=jnp.int32) % (NP - N)
    src = jnp.concatenate([ei[0], pad_rows])
    dst = jnp.concatenate([ei[1], pad_rows])
    srcm = src.reshape(CHUNKS, CH)
    dstm = dst.reshape(CHUNKS, CH)
    x_p = jnp.pad(x, ((0, NP - N), (0, 0)))
    b0r = b0.reshape(1, D)
    b1r = b1.reshape(1, D)
    b2r = b2.reshape(1, D)

    dall = _sc_degree(dst)
    g0, dv = _tc_first(x_p, W0, dall)

    a = _sc_aggregate(g0, srcm, dstm)
    g1 = _tc_mid(a, g0, dv, b0r, W1)

    a = _sc_aggregate(g1, srcm, dstm)
    g2 = _tc_mid(a, g1, dv, b1r, W2)

    a = _sc_aggregate(g2, srcm, dstm)
    return _tc_last(a, g2, dv, b2r)
